# Initial kernel scaffold; baseline (speedup 1.0000x reference)
import jax
import jax.numpy as jnp
from jax.experimental import pallas as pl


def kernel(ids, ori_weight, think_weight):
    raise NotImplementedError("write your pallas kernel here")


# speedup vs baseline: 6.5656x; 6.5656x over previous
import functools

import jax
import jax.numpy as jnp
from jax import lax
from jax.experimental import pallas as pl
from jax.experimental.pallas import tpu as pltpu
from jax.experimental.pallas import tpu_sc as plsc

ORI_N = 100000
THINK_N = 100000
EMBED_D = 64

NC = 2
NS = 16
NW = NC * NS

NB = 512


def _gather_kernel(B):
    C = B // NW
    assert C % NB == 0
    mesh = plsc.VectorSubcoreMesh(core_axis_name="c", subcore_axis_name="s")

    @functools.partial(
        pl.kernel,
        out_type=jax.ShapeDtypeStruct((B, EMBED_D), jnp.float32),
        mesh=mesh,
        scratch_types=[
            pltpu.VMEM((NB,), jnp.int32),
            pltpu.VMEM((NB, EMBED_D), jnp.float32),
            pltpu.SemaphoreType.DMA,
        ],
        compiler_params=pltpu.CompilerParams(use_tc_tiling_on_sc=False),
    )
    def k(ids_hbm, table_hbm, out_hbm, idx_v, rows_v, sem):
        wid = lax.axis_index("s") * NC + lax.axis_index("c")
        base = wid * C

        def body(t, carry):
            off = base + t * NB
            pltpu.sync_copy(ids_hbm.at[pl.ds(off, NB)], idx_v)
            pltpu.async_copy(table_hbm.at[idx_v], rows_v, sem).wait()
            pltpu.sync_copy(rows_v, out_hbm.at[pl.ds(off, NB)])
            return carry

        lax.fori_loop(0, C // NB, body, 0)

    return k


def kernel(ids, ori_weight, think_weight):
    table = jnp.concatenate([ori_weight, think_weight], axis=0)
    ids_flat = ids.reshape(-1).astype(jnp.int32)
    out = _gather_kernel(ids_flat.shape[0])(ids_flat, table)
    return out.reshape(ids.shape + (EMBED_D,))


# speedup vs baseline: 7.0121x; 1.0680x over previous
import functools

import jax
import jax.numpy as jnp
from jax import lax
from jax.experimental import pallas as pl
from jax.experimental.pallas import tpu as pltpu
from jax.experimental.pallas import tpu_sc as plsc

ORI_N = 100000
THINK_N = 100000
EMBED_D = 64

NC = 2
NS = 16
NW = NC * NS

NB = 512


def _gather_kernel(B):
    C = B // NW
    assert C % NB == 0
    mesh = plsc.VectorSubcoreMesh(core_axis_name="c", subcore_axis_name="s")

    T = C // NB

    @functools.partial(
        pl.kernel,
        out_type=jax.ShapeDtypeStruct((B, EMBED_D), jnp.float32),
        mesh=mesh,
        scratch_types=[
            pltpu.VMEM((2, NB), jnp.int32),
            pltpu.VMEM((2, NB, EMBED_D), jnp.float32),
            [pltpu.SemaphoreType.DMA] * 2,
            [pltpu.SemaphoreType.DMA] * 2,
            [pltpu.SemaphoreType.DMA] * 2,
        ],
        compiler_params=pltpu.CompilerParams(use_tc_tiling_on_sc=False),
    )
    def k(ids_hbm, table_hbm, out_hbm, idx_v, rows_v, idsem, gsem, outsem):
        wid = lax.axis_index("s") * NC + lax.axis_index("c")
        base = wid * C

        def idload(t, b):
            pltpu.async_copy(ids_hbm.at[pl.ds(base + t * NB, NB)],
                             idx_v.at[b], idsem[b])

        def gather(t, b):
            del t
            pltpu.async_copy(table_hbm.at[idx_v.at[b]], rows_v.at[b], gsem[b])

        def outcopy(t, b):
            pltpu.async_copy(rows_v.at[b],
                             out_hbm.at[pl.ds(base + t * NB, NB)], outsem[b])

        def wait_id(b):
            pltpu.make_async_copy(ids_hbm.at[pl.ds(0, NB)], idx_v.at[b],
                                  idsem[b]).wait()

        def wait_g(b):
            pltpu.make_async_copy(table_hbm.at[pl.ds(0, NB)], rows_v.at[b],
                                  gsem[b]).wait()

        def wait_out(b):
            pltpu.make_async_copy(rows_v.at[b], out_hbm.at[pl.ds(0, NB)],
                                  outsem[b]).wait()

        idload(0, 0)
        idload(1, 1)
        wait_id(0)
        gather(0, 0)

        def body(i, carry):
            for s in range(2):
                t = 2 * i + 1 + s
                b = (1 + s) % 2
                ob = 1 - b

                @pl.when(t < T)
                def _():
                    @pl.when(t >= 2)
                    def _():
                        wait_out(b)

                    wait_id(b)
                    gather(t, b)
                    wait_g(ob)

                    @pl.when(t + 1 < T)
                    def _():
                        idload(t + 1, ob)

                    outcopy(t - 1, ob)

            return carry

        lax.fori_loop(0, T // 2, body, 0)

        lb = (T - 1) % 2
        wait_g(lb)
        outcopy(T - 1, lb)
        wait_out(1 - lb)
        wait_out(lb)

    return k


def kernel(ids, ori_weight, think_weight):
    table = jnp.concatenate([ori_weight, think_weight], axis=0)
    ids_flat = ids.reshape(-1).astype(jnp.int32)
    out = _gather_kernel(ids_flat.shape[0])(ids_flat, table)
    return out.reshape(ids.shape + (EMBED_D,))
